# Initial kernel scaffold; baseline (speedup 1.0000x reference)
import jax
import jax.numpy as jnp
from jax.experimental import pallas as pl


def kernel(hidden_states, router_weight, router_bias, gate_up_proj, gate_up_proj_bias, down_proj, down_proj_bias):
    raise NotImplementedError("write your pallas kernel here")


# speedup vs baseline: 10.7510x; 10.7510x over previous
import functools

import jax
import jax.numpy as jnp
from jax import lax
from jax.experimental import pallas as pl
from jax.experimental.pallas import tpu as pltpu
from jax.experimental.pallas import tpu_sc as plsc

_E = 16
_T = 2048
_D = 1024
_TB = 128
_NB = _T // _TB + _E
_LOG2_TB = 7
_NW = 32
_RPW = _T // _NW
_LIMIT = 7.0
_ALPHA = 1.702


def _router_body(x_ref, rw_ref, rb_ref, scores_ref, pos_ref, bexp_ref, bval_ref):
    x = x_ref[...]
    rw = rw_ref[...]
    logits = lax.dot_general(x, rw, (((1,), (1,)), ((), ())),
                             preferred_element_type=jnp.float32)
    logits = logits + rb_ref[...]
    m = jnp.max(logits, axis=1, keepdims=True)
    eidx = lax.broadcasted_iota(jnp.int32, (_T, _E), 1)
    sel = jnp.where(logits == m, eidx, _E)
    e_t = jnp.min(sel, axis=1, keepdims=True)
    onehot = (eidx == e_t).astype(jnp.int32)
    scores_ref[...] = onehot.astype(jnp.float32)

    cum = onehot
    sh = 1
    while sh < _T:
        z = jnp.zeros((sh, _E), jnp.int32)
        cum = cum + jnp.concatenate([z, cum[: _T - sh]], axis=0)
        sh *= 2
    counts = cum[_T - 1:_T, :]
    bpe = lax.shift_right_logical(counts + (_TB - 1), _LOG2_TB)
    ic = bpe
    sh = 1
    while sh < _E:
        z = jnp.zeros((1, sh), jnp.int32)
        ic = ic + jnp.concatenate([z, ic[:, : _E - sh]], axis=1)
        sh *= 2
    total = ic[:, _E - 1:_E]
    bstart = ic - bpe
    pad_off = lax.shift_left(bstart, _LOG2_TB)
    posmat = onehot * (pad_off + cum - 1)
    pos_ref[...] = jnp.sum(posmat, axis=1, keepdims=True)

    bi = lax.broadcasted_iota(jnp.int32, (_NB, _E), 0)
    ieff = jnp.minimum(bi, total - 1)
    cmp = (bstart <= ieff).astype(jnp.int32)
    bexp_ref[...] = jnp.sum(cmp, axis=1, keepdims=True) - 1
    bval_ref[...] = (lax.broadcasted_iota(jnp.int32, (_NB, 1), 0)
                     < total).astype(jnp.int32)


def _run_router(x, rw, rb):
    return pl.pallas_call(
        _router_body,
        out_shape=(
            jax.ShapeDtypeStruct((_T, _E), jnp.float32),
            jax.ShapeDtypeStruct((_T, 1), jnp.int32),
            jax.ShapeDtypeStruct((_NB, 1), jnp.int32),
            jax.ShapeDtypeStruct((_NB, 1), jnp.int32),
        ),
    )(x, rw, rb.reshape(1, _E))


@functools.cache
def _sc_kernels():
    mesh = plsc.VectorSubcoreMesh(core_axis_name="c", subcore_axis_name="s")
    scratch = [
        pltpu.VMEM((_RPW,), jnp.int32),
        pltpu.VMEM((_RPW, _D), jnp.float32),
        pltpu.SemaphoreType.DMA,
    ]

    @functools.partial(
        pl.kernel,
        out_type=jax.ShapeDtypeStruct((_NB * _TB, _D), jnp.float32),
        mesh=mesh,
        scratch_types=scratch,
    )
    def scatter(x_hbm, pos_hbm, xs_hbm, idx_v, rows_v, sem):
        wid = lax.axis_index("s") * 2 + lax.axis_index("c")
        base = wid * _RPW
        pltpu.sync_copy(pos_hbm.at[pl.ds(base, _RPW)], idx_v)
        pltpu.sync_copy(x_hbm.at[pl.ds(base, _RPW)], rows_v)
        pltpu.async_copy(rows_v, xs_hbm.at[idx_v], sem).wait()

    @functools.partial(
        pl.kernel,
        out_type=jax.ShapeDtypeStruct((_T, _D), jnp.float32),
        mesh=mesh,
        scratch_types=scratch,
    )
    def gather(os_hbm, pos_hbm, out_hbm, idx_v, rows_v, sem):
        wid = lax.axis_index("s") * 2 + lax.axis_index("c")
        base = wid * _RPW
        pltpu.sync_copy(pos_hbm.at[pl.ds(base, _RPW)], idx_v)
        pltpu.async_copy(os_hbm.at[idx_v], rows_v, sem).wait()
        pltpu.sync_copy(rows_v, out_hbm.at[pl.ds(base, _RPW)])

    return scatter, gather


def _sc_scatter(x, pos):
    return _sc_kernels()[0](x, pos)


def _sc_gather(out_sorted, pos):
    return _sc_kernels()[1](out_sorted, pos)


def _expert_body(bexp_ref, bval_ref, xs_ref, gup_ref, gupb_ref, dw_ref,
                 dwb_ref, out_ref):
    i = pl.program_id(0)

    @pl.when(bval_ref[i] == 1)
    def _():
        xb = xs_ref[...]
        guT = lax.dot_general(gup_ref[0], xb, (((0,), (1,)), ((), ())),
                              preferred_element_type=jnp.float32)
        guT = guT + gupb_ref[0]
        gu3 = guT.reshape(_D, 2, _TB)
        gateT = gu3[:, 0, :]
        upT = gu3[:, 1, :]
        gateT = jnp.minimum(gateT, _LIMIT)
        upT = jnp.clip(upT, -_LIMIT, _LIMIT)
        gluT = gateT / (1.0 + jnp.exp(gateT * (-_ALPHA)))
        interT = (upT + 1.0) * gluT
        ob = lax.dot_general(interT, dw_ref[0], (((0,), (0,)), ((), ())),
                             preferred_element_type=jnp.float32)
        out_ref[...] = ob + dwb_ref[0]


def _run_experts(bexp, bval, x_sorted, gup, gupb, dw, dwb):
    grid_spec = pltpu.PrefetchScalarGridSpec(
        num_scalar_prefetch=2,
        grid=(_NB,),
        in_specs=[
            pl.BlockSpec((_TB, _D), lambda i, be, bv: (i, 0)),
            pl.BlockSpec((1, _D, 2 * _D), lambda i, be, bv: (be[i], 0, 0)),
            pl.BlockSpec((1, 2 * _D, 1), lambda i, be, bv: (be[i], 0, 0)),
            pl.BlockSpec((1, _D, _D), lambda i, be, bv: (be[i], 0, 0)),
            pl.BlockSpec((1, 1, _D), lambda i, be, bv: (be[i], 0, 0)),
        ],
        out_specs=pl.BlockSpec((_TB, _D), lambda i, be, bv: (i, 0)),
    )
    return pl.pallas_call(
        _expert_body,
        grid_spec=grid_spec,
        out_shape=jax.ShapeDtypeStruct((_NB * _TB, _D), jnp.float32),
    )(bexp, bval, x_sorted, gup, gupb.reshape(_E, 2 * _D, 1), dw,
      dwb.reshape(_E, 1, _D))


def kernel(hidden_states, router_weight, router_bias, gate_up_proj,
           gate_up_proj_bias, down_proj, down_proj_bias):
    b, s, d = hidden_states.shape
    x = hidden_states.reshape(b * s, d)
    scores, pos2, bexp2, bval2 = _run_router(x, router_weight, router_bias)
    pos = pos2.reshape(_T)
    bexp = bexp2.reshape(_NB)
    bval = bval2.reshape(_NB)
    x_sorted = _sc_scatter(x, pos)
    out_sorted = _run_experts(bexp, bval, x_sorted, gate_up_proj,
                              gate_up_proj_bias, down_proj, down_proj_bias)
    out = _sc_gather(out_sorted, pos)
    return out.reshape(b, s, d), scores
